# baseline (speedup 1.0000x reference)
import functools

import jax
import jax.numpy as jnp
from jax import lax
from jax.experimental import pallas as pl
from jax.experimental.pallas import tpu as pltpu
from jax.experimental.pallas import tpu_sc as plsc

D = 32
T = 200
NC = 2
NS = 16
NW = NC * NS
LANES = 16
SKEW = 129

TT = T // 8
BT = 4096 // 128


@jax.jit
def _emb(idx_native, tok_padded, pos_table):
    mesh = plsc.VectorSubcoreMesh(core_axis_name="c", subcore_axis_name="s")

    @functools.partial(
        pl.kernel,
        out_type=jax.ShapeDtypeStruct((T, D // 8, BT, 8, 128), jnp.float32),
        mesh=mesh,
        scratch_types=[
            pltpu.VMEM((2048,), jnp.int32),
            pltpu.VMEM((2048, D), jnp.float32),
            pltpu.VMEM((32, 8, SKEW), jnp.float32),
            pltpu.VMEM((T, D), jnp.float32),
            pltpu.SemaphoreType.DMA,
            pltpu.SemaphoreType.DMA,
            pltpu.SemaphoreType.DMA,
        ],
        compiler_params=pltpu.CompilerParams(
            use_tc_tiling_on_sc=False, needs_layout_passes=False
        ),
    )
    def body(idx_hbm, tok_hbm, pos_hbm, out_hbm,
             idx_v, rows_v, tsk_v, pos_v, gsem, ssem, isem):
        w = lax.axis_index("s") * NC + lax.axis_index("c")
        pltpu.sync_copy(pos_hbm.at[pl.ds(0, T)], pos_v)

        lane = lax.iota(jnp.int32, LANES)
        ct_l = lane >> 3
        ci_l = lane & 7

        def load_idx(tt, bank):
            pltpu.async_copy(
                idx_hbm.at[pl.ds((tt * BT + w) * 1024, 1024)],
                idx_v.at[pl.ds(bank * 1024, 1024)],
                isem,
            )

        def fire_gathers(bank):
            for ti in range(8):
                pltpu.async_copy(
                    tok_hbm.at[idx_v.at[pl.ds(bank * 1024 + ti * 128, 128)]],
                    rows_v.at[pl.ds(bank * 1024 + ti * 128, 128)],
                    gsem,
                )

        load_idx(0, 0)
        pltpu.make_async_copy(
            idx_hbm.at[pl.ds(0, 1024)], idx_v.at[pl.ds(0, 1024)], isem
        ).wait()
        fire_gathers(0)
        load_idx(1, 1)

        def tt_body(tt, carry):
            par = lax.rem(tt, 2)
            nxt = 1 - par

            @pl.when(tt + 1 < TT)
            def _():
                pltpu.make_async_copy(
                    idx_hbm.at[pl.ds(0, 1024)],
                    idx_v.at[pl.ds(nxt * 1024, 1024)],
                    isem,
                ).wait()
                fire_gathers(nxt)

            for h in range(2):
                for ti in range(4 * h, 4 * h + 4):
                    pltpu.make_async_copy(
                        tok_hbm.at[
                            idx_v.at[pl.ds(par * 1024 + ti * 128, 128)]
                        ],
                        rows_v.at[pl.ds(par * 1024 + ti * 128, 128)],
                        gsem,
                    ).wait()
                if h == 1:
                    @pl.when(tt + 2 < TT)
                    def _():
                        load_idx(tt + 2, par)
                hidx = tt * 2 + h
                hp16 = lax.rem(hidx, 2) * 16

                @pl.when(hidx >= 2)
                def _(hp16=hp16, h=h):
                    for k in range(4):
                        pltpu.make_async_copy(
                            tsk_v.at[pl.ds(hp16 + k * 4, 4), :,
                                     pl.ds(0, 128)],
                            out_hbm.at[tt * 8 + h * 4 + k, :, w],
                            ssem,
                        ).wait()

                def k_body(k, carry2, h=h, hp16=hp16):
                    t = tt * 8 + h * 4 + k
                    rbase = par * 1024 + (h * 4 + k) * 128
                    p0 = pos_v[t, pl.ds(0, LANES)]
                    p1 = pos_v[t, pl.ds(LANES, LANES)]
                    kct0 = ct_l + (k * 4) + hp16
                    kct1 = kct0 + 2

                    def bi_body(bi, c3, rbase=rbase, p0=p0, p1=p1,
                                kct0=kct0, kct1=kct1):
                        bi_vec = jnp.full((LANES,), bi, jnp.int32)
                        v0 = rows_v[rbase + bi, pl.ds(0, LANES)] + p0
                        plsc.store_scatter(tsk_v, [kct0, ci_l, bi_vec], v0)
                        v1 = rows_v[rbase + bi, pl.ds(LANES, LANES)] + p1
                        plsc.store_scatter(tsk_v, [kct1, ci_l, bi_vec], v1)
                        return c3

                    lax.fori_loop(0, 128, bi_body, 0, unroll=16)
                    return carry2

                lax.fori_loop(0, 4, k_body, 0)

                for k in range(4):
                    t = tt * 8 + h * 4 + k
                    pltpu.async_copy(
                        tsk_v.at[pl.ds(hp16 + k * 4, 4), :, pl.ds(0, 128)],
                        out_hbm.at[t, :, w],
                        ssem,
                    )
            return carry

        lax.fori_loop(0, TT, tt_body, 0)

        for hp in range(2):
            for k in range(4):
                pltpu.make_async_copy(
                    tsk_v.at[pl.ds(hp * 16 + k * 4, 4), :, pl.ds(0, 128)],
                    out_hbm.at[(TT - 1) * 8 + hp * 4 + k, :, w],
                    ssem,
                ).wait()

    return body(idx_native, tok_padded, pos_table)


def kernel(idx, tok_table, pos_table):
    idx = idx.astype(jnp.int32)
    idx_native = (
        idx.T.reshape(TT, 8, BT, 128).transpose(0, 2, 1, 3).reshape(-1)
    )
    tok_lin = lax.optimization_barrier(tok_table.reshape(-1))
    q = _emb(idx_native, tok_lin.reshape(-1, D), pos_table)
    return q.transpose(2, 4, 0, 1, 3).reshape(4096, T, D)


# speedup vs baseline: 1.0039x; 1.0039x over previous
import functools

import jax
import jax.numpy as jnp
from jax import lax
from jax.experimental import pallas as pl
from jax.experimental.pallas import tpu as pltpu
from jax.experimental.pallas import tpu_sc as plsc

D = 32
T = 200
NC = 2
NS = 16
NW = NC * NS
LANES = 16
SKEW = 129

TT = T // 8
BT = 4096 // 128


@jax.jit
def _emb(idx_native, tok_padded, pos_table):
    mesh = plsc.VectorSubcoreMesh(core_axis_name="c", subcore_axis_name="s")

    @functools.partial(
        pl.kernel,
        out_type=jax.ShapeDtypeStruct((T, D // 8, BT, 8, 128), jnp.float32),
        mesh=mesh,
        scratch_types=[
            pltpu.VMEM((2048,), jnp.int32),
            pltpu.VMEM((2048, D), jnp.float32),
            pltpu.VMEM((32, 8, SKEW), jnp.float32),
            pltpu.VMEM((T, D), jnp.float32),
            pltpu.SemaphoreType.DMA,
            pltpu.SemaphoreType.DMA,
            pltpu.SemaphoreType.DMA,
        ],
        compiler_params=pltpu.CompilerParams(
            use_tc_tiling_on_sc=False, needs_layout_passes=False
        ),
    )
    def body(idx_hbm, tok_hbm, pos_hbm, out_hbm,
             idx_v, rows_v, tsk_v, pos_v, gsem, ssem, isem):
        w = lax.axis_index("s") * NC + lax.axis_index("c")
        pltpu.sync_copy(pos_hbm.at[pl.ds(0, T)], pos_v)

        lane = lax.iota(jnp.int32, LANES)
        ct_l = lane >> 3
        ci_l = lane & 7

        def load_idx(tt, bank):
            pltpu.async_copy(
                idx_hbm.at[pl.ds((tt * BT + w) * 1024, 1024)],
                idx_v.at[pl.ds(bank * 1024, 1024)],
                isem,
            )

        def fire_gathers(bank):
            for ti in range(8):
                pltpu.async_copy(
                    tok_hbm.at[idx_v.at[pl.ds(bank * 1024 + ti * 128, 128)]],
                    rows_v.at[pl.ds(bank * 1024 + ti * 128, 128)],
                    gsem,
                )

        load_idx(0, 0)
        pltpu.make_async_copy(
            idx_hbm.at[pl.ds(0, 1024)], idx_v.at[pl.ds(0, 1024)], isem
        ).wait()
        fire_gathers(0)
        load_idx(1, 1)

        def tt_body(tt, carry):
            par = lax.rem(tt, 2)
            nxt = 1 - par

            @pl.when(tt + 1 < TT)
            def _():
                pltpu.make_async_copy(
                    idx_hbm.at[pl.ds(0, 1024)],
                    idx_v.at[pl.ds(nxt * 1024, 1024)],
                    isem,
                ).wait()
                fire_gathers(nxt)

            for h in range(2):
                for ti in range(4 * h, 4 * h + 4):
                    pltpu.make_async_copy(
                        tok_hbm.at[
                            idx_v.at[pl.ds(par * 1024 + ti * 128, 128)]
                        ],
                        rows_v.at[pl.ds(par * 1024 + ti * 128, 128)],
                        gsem,
                    ).wait()
                if h == 1:
                    @pl.when(tt + 2 < TT)
                    def _():
                        load_idx(tt + 2, par)
                hidx = tt * 2 + h
                hp16 = lax.rem(hidx, 2) * 16

                @pl.when(hidx >= 2)
                def _(hp16=hp16, h=h):
                    for k in range(4):
                        pltpu.make_async_copy(
                            tsk_v.at[pl.ds(hp16 + k * 4, 4), :,
                                     pl.ds(0, 128)],
                            out_hbm.at[tt * 8 + h * 4 + k, :, w],
                            ssem,
                        ).wait()

                def k_body(k, carry2, h=h, hp16=hp16):
                    t = tt * 8 + h * 4 + k
                    rbase = par * 1024 + (h * 4 + k) * 128
                    p0 = pos_v[t, pl.ds(0, LANES)]
                    p1 = pos_v[t, pl.ds(LANES, LANES)]
                    kct0 = ct_l + (k * 4) + hp16
                    kct1 = kct0 + 2

                    def bi_body(bi, c3, rbase=rbase, p0=p0, p1=p1,
                                kct0=kct0, kct1=kct1):
                        bi_vec = jnp.full((LANES,), bi, jnp.int32)
                        v0 = rows_v[rbase + bi, pl.ds(0, LANES)] + p0
                        plsc.store_scatter(tsk_v, [kct0, ci_l, bi_vec], v0)
                        v1 = rows_v[rbase + bi, pl.ds(LANES, LANES)] + p1
                        plsc.store_scatter(tsk_v, [kct1, ci_l, bi_vec], v1)
                        return c3

                    lax.fori_loop(0, 128, bi_body, 0, unroll=8)
                    return carry2

                lax.fori_loop(0, 4, k_body, 0)

                for k in range(4):
                    t = tt * 8 + h * 4 + k
                    pltpu.async_copy(
                        tsk_v.at[pl.ds(hp16 + k * 4, 4), :, pl.ds(0, 128)],
                        out_hbm.at[t, :, w],
                        ssem,
                    )
            return carry

        lax.fori_loop(0, TT, tt_body, 0)

        for hp in range(2):
            for k in range(4):
                pltpu.make_async_copy(
                    tsk_v.at[pl.ds(hp * 16 + k * 4, 4), :, pl.ds(0, 128)],
                    out_hbm.at[(TT - 1) * 8 + hp * 4 + k, :, w],
                    ssem,
                ).wait()

    return body(idx_native, tok_padded, pos_table)


def kernel(idx, tok_table, pos_table):
    idx = idx.astype(jnp.int32)
    idx_native = (
        idx.T.reshape(TT, 8, BT, 128).transpose(0, 2, 1, 3).reshape(-1)
    )
    tok_lin = lax.optimization_barrier(tok_table.reshape(-1))
    q = _emb(idx_native, tok_lin.reshape(-1, D), pos_table)
    return q.transpose(2, 4, 0, 1, 3).reshape(4096, T, D)


# speedup vs baseline: 1.3262x; 1.3210x over previous
import functools

import jax
import jax.numpy as jnp
from jax import lax
from jax.experimental import pallas as pl
from jax.experimental.pallas import tpu as pltpu
from jax.experimental.pallas import tpu_sc as plsc

D = 32
T = 200
NC = 2
NS = 16
NW = NC * NS
LANES = 16
SKEW = 129

TT = T // 8
BT = 4096 // 128


@jax.jit
def _emb(idx_native, tok_padded, pos_table):
    mesh = plsc.VectorSubcoreMesh(core_axis_name="c", subcore_axis_name="s")

    @functools.partial(
        pl.kernel,
        out_type=jax.ShapeDtypeStruct((T, D // 8, BT, 8, 128), jnp.float32),
        mesh=mesh,
        scratch_types=[
            pltpu.VMEM((2048,), jnp.int32),
            pltpu.VMEM((2048, D), jnp.float32),
            pltpu.VMEM((32, 8, SKEW), jnp.float32),
            pltpu.VMEM((T, D), jnp.float32),
            pltpu.SemaphoreType.DMA,
            pltpu.SemaphoreType.DMA,
            pltpu.SemaphoreType.DMA,
        ],
        compiler_params=pltpu.CompilerParams(
            use_tc_tiling_on_sc=False, needs_layout_passes=False
        ),
    )
    def body(idx_hbm, tok_hbm, pos_hbm, out_hbm,
             idx_v, rows_v, tsk_v, pos_v, gsem, ssem, isem):
        w = lax.axis_index("s") * NC + lax.axis_index("c")
        pltpu.sync_copy(pos_hbm.at[pl.ds(0, T)], pos_v)

        lane = lax.iota(jnp.int32, LANES)
        ct_l = lane >> 3
        ci_l = lane & 7

        def load_idx(tt, bank):
            pltpu.async_copy(
                idx_hbm.at[pl.ds((tt * BT + w) * 1024, 1024)],
                idx_v.at[pl.ds(bank * 1024, 1024)],
                isem,
            )

        def fire_gathers(bank):
            for ti in range(8):
                pltpu.async_copy(
                    tok_hbm.at[idx_v.at[pl.ds(bank * 1024 + ti * 128, 128)]],
                    rows_v.at[pl.ds(bank * 1024 + ti * 128, 128)],
                    gsem,
                )

        load_idx(0, 0)
        pltpu.make_async_copy(
            idx_hbm.at[pl.ds(0, 1024)], idx_v.at[pl.ds(0, 1024)], isem
        ).wait()
        fire_gathers(0)
        load_idx(1, 1)

        def tt_body(tt, carry):
            par = lax.rem(tt, 2)
            nxt = 1 - par

            @pl.when(tt + 1 < TT)
            def _():
                pltpu.make_async_copy(
                    idx_hbm.at[pl.ds(0, 1024)],
                    idx_v.at[pl.ds(nxt * 1024, 1024)],
                    isem,
                ).wait()
                fire_gathers(nxt)

            for h in range(2):
                for ti in range(4 * h, 4 * h + 4):
                    pltpu.make_async_copy(
                        tok_hbm.at[
                            idx_v.at[pl.ds(par * 1024 + ti * 128, 128)]
                        ],
                        rows_v.at[pl.ds(par * 1024 + ti * 128, 128)],
                        gsem,
                    ).wait()
                if h == 1:
                    @pl.when(tt + 2 < TT)
                    def _():
                        load_idx(tt + 2, par)
                hidx = tt * 2 + h
                hp16 = lax.rem(hidx, 2) * 16

                @pl.when(hidx >= 2)
                def _(hp16=hp16, h=h):
                    for k in range(4):
                        pltpu.make_async_copy(
                            tsk_v.at[pl.ds(hp16 + k * 4, 4), :,
                                     pl.ds(0, 128)],
                            out_hbm.at[tt * 8 + h * 4 + k, :, w],
                            ssem,
                        ).wait()

                def k_body(k, carry2, h=h, hp16=hp16):
                    t = tt * 8 + h * 4 + k
                    rbase = par * 1024 + (h * 4 + k) * 128
                    p0 = pos_v[t, pl.ds(0, LANES)]
                    p1 = pos_v[t, pl.ds(LANES, LANES)]
                    kct0 = ct_l + (k * 4) + hp16
                    kct1 = kct0 + 2

                    @plsc.parallel_loop(0, 128, unroll=8)
                    def _(bi, rbase=rbase, p0=p0, p1=p1,
                          kct0=kct0, kct1=kct1):
                        bi_vec = jnp.full((LANES,), bi, jnp.int32)
                        v0 = rows_v[rbase + bi, pl.ds(0, LANES)] + p0
                        plsc.store_scatter(tsk_v, [kct0, ci_l, bi_vec], v0)
                        v1 = rows_v[rbase + bi, pl.ds(LANES, LANES)] + p1
                        plsc.store_scatter(tsk_v, [kct1, ci_l, bi_vec], v1)

                    return carry2

                lax.fori_loop(0, 4, k_body, 0)

                for k in range(4):
                    t = tt * 8 + h * 4 + k
                    pltpu.async_copy(
                        tsk_v.at[pl.ds(hp16 + k * 4, 4), :, pl.ds(0, 128)],
                        out_hbm.at[t, :, w],
                        ssem,
                    )
            return carry

        lax.fori_loop(0, TT, tt_body, 0)

        for hp in range(2):
            for k in range(4):
                pltpu.make_async_copy(
                    tsk_v.at[pl.ds(hp * 16 + k * 4, 4), :, pl.ds(0, 128)],
                    out_hbm.at[(TT - 1) * 8 + hp * 4 + k, :, w],
                    ssem,
                ).wait()

    return body(idx_native, tok_padded, pos_table)


def kernel(idx, tok_table, pos_table):
    idx = idx.astype(jnp.int32)
    idx_native = (
        idx.T.reshape(TT, 8, BT, 128).transpose(0, 2, 1, 3).reshape(-1)
    )
    tok_lin = lax.optimization_barrier(tok_table.reshape(-1))
    q = _emb(idx_native, tok_lin.reshape(-1, D), pos_table)
    return q.transpose(2, 4, 0, 1, 3).reshape(4096, T, D)
